# initial kernel scaffold (unmeasured)
import jax
import jax.numpy as jnp
from jax import lax
from jax.experimental import pallas as pl
from jax.experimental.pallas import tpu as pltpu


def kernel(
    x,
):
    def body(*refs):
        pass

    out_shape = jax.ShapeDtypeStruct(..., jnp.float32)
    return pl.pallas_call(body, out_shape=out_shape)(...)


# baseline (device time: 85292 ns/iter reference)
import jax
import jax.numpy as jnp
from jax import lax
from jax.experimental import pallas as pl
from jax.experimental.pallas import tpu as pltpu

P = 4
N_HOP = P - 1


def kernel(x):
    _, m, n_glob = x.shape
    n = n_glob // P

    def body(x_ref, out_ref, send_buf, recv_buf, send_sems, recv_sems):
        my_x = lax.axis_index("x")
        my_y = lax.axis_index("y")
        my_z = lax.axis_index("z")
        right = lax.rem(my_y + 1, P)
        left = lax.rem(my_y + P - 1, P)

        barrier_sem = pltpu.get_barrier_semaphore()
        for nbr in (left, right):
            pl.semaphore_signal(
                barrier_sem, inc=1,
                device_id=(my_x, nbr, my_z),
                device_id_type=pl.DeviceIdType.MESH,
            )
        pl.semaphore_wait(barrier_sem, 2)

        def chunk(c):
            return x_ref[0, :, pl.ds(c * n, n)]

        for h in range(N_HOP):
            c = lax.rem(my_y - h - 1 + 2 * P, P)
            if h == 0:
                send_buf[h] = chunk(c).astype(jnp.bfloat16)
            else:
                send_buf[h] = (
                    recv_buf[h - 1].astype(jnp.float32) + chunk(c)
                ).astype(jnp.bfloat16)
            rdma = pltpu.make_async_remote_copy(
                src_ref=send_buf.at[h],
                dst_ref=recv_buf.at[h],
                send_sem=send_sems.at[h],
                recv_sem=recv_sems.at[h],
                device_id=(my_x, right, my_z),
                device_id_type=pl.DeviceIdType.MESH,
            )
            rdma.start()
            rdma.wait()

        out_ref[:, :] = recv_buf[N_HOP - 1].astype(jnp.float32) + chunk(my_y)

    return pl.pallas_call(
        body,
        out_shape=jax.ShapeDtypeStruct((m, n), jnp.float32),
        in_specs=[pl.BlockSpec(memory_space=pltpu.VMEM)],
        out_specs=pl.BlockSpec(memory_space=pltpu.VMEM),
        scratch_shapes=[
            pltpu.VMEM((N_HOP, m, n), jnp.bfloat16),
            pltpu.VMEM((N_HOP, m, n), jnp.bfloat16),
            pltpu.SemaphoreType.DMA((N_HOP,)),
            pltpu.SemaphoreType.DMA((N_HOP,)),
        ],
        compiler_params=pltpu.CompilerParams(collective_id=0),
    )(x)


# device time: 45982 ns/iter; 1.8549x vs baseline; 1.8549x over previous
import functools

import jax
import jax.numpy as jnp
from jax import lax
from jax.experimental import pallas as pl
from jax.experimental.pallas import tpu as pltpu

PY = 4
NXZ = 8
YHOP = PY - 1
FHOP = 4
BHOP = 3


def _ring_coords(p):
    p = lax.rem(p + NXZ, NXZ)
    x = jnp.where(p < 4, 0, 1)
    z = jnp.where(p < 4, p, 7 - p)
    return x, z


def kernel(x):
    _, m, n_glob = x.shape
    n = n_glob // PY
    rb = m // NXZ

    def body(x_ref, out_ref, ys_buf, yr_buf, fwd_buf, bwd_buf,
             fs_sem, fr_sem, bs_sem, br_sem):
        mx = lax.axis_index("x")
        my = lax.axis_index("y")
        mz = lax.axis_index("z")
        y_right = lax.rem(my + 1, PY)
        y_left = lax.rem(my + PY - 1, PY)
        p = jnp.where(mx == 0, mz, 7 - mz)
        rx, rz = _ring_coords(p + 1)
        lx, lz = _ring_coords(p - 1)

        barrier_sem = pltpu.get_barrier_semaphore()
        for dev in ((mx, y_left, mz), (mx, y_right, mz),
                    (lx, my, lz), (rx, my, rz)):
            pl.semaphore_signal(
                barrier_sem, inc=1,
                device_id=dev, device_id_type=pl.DeviceIdType.MESH,
            )
        pl.semaphore_wait(barrier_sem, 4)

        def chunk(c):
            return x_ref[0, pl.ds(p * rb, rb), pl.ds(c * n, n)]

        for h in range(YHOP):
            c = lax.rem(my - h - 1 + 2 * PY, PY)
            if h == 0:
                ys_buf[h] = chunk(c).astype(jnp.bfloat16)
            else:
                ys_buf[h] = (
                    yr_buf[h - 1].astype(jnp.float32) + chunk(c)
                ).astype(jnp.bfloat16)
            rdma = pltpu.make_async_remote_copy(
                src_ref=ys_buf.at[h],
                dst_ref=yr_buf.at[h],
                send_sem=bs_sem.at[h],
                recv_sem=br_sem.at[h],
                device_id=(mx, y_right, mz),
                device_id_type=pl.DeviceIdType.MESH,
            )
            rdma.start()
            rdma.wait()

        own = yr_buf[YHOP - 1].astype(jnp.float32) + chunk(my)
        out_ref[pl.ds(p * rb, rb), :] = own
        own_bf = own.astype(jnp.bfloat16)
        fwd_buf[0] = own_bf
        bwd_buf[0] = own_bf

        @functools.partial(pl.run_scoped, mid_sem=pltpu.SemaphoreType.REGULAR)
        def _(mid_sem):
            for dev in ((lx, my, lz), (rx, my, rz)):
                pl.semaphore_signal(
                    mid_sem, inc=1,
                    device_id=dev, device_id_type=pl.DeviceIdType.MESH,
                )
            pl.semaphore_wait(mid_sem, 2)

        for h in range(FHOP):
            fwd = pltpu.make_async_remote_copy(
                src_ref=fwd_buf.at[h],
                dst_ref=fwd_buf.at[h + 1],
                send_sem=fs_sem.at[h],
                recv_sem=fr_sem.at[h],
                device_id=(rx, my, rz),
                device_id_type=pl.DeviceIdType.MESH,
            )
            fwd.start()
            if h < BHOP:
                bwd = pltpu.make_async_remote_copy(
                    src_ref=bwd_buf.at[h],
                    dst_ref=bwd_buf.at[h + 1],
                    send_sem=bs_sem.at[h],
                    recv_sem=br_sem.at[h],
                    device_id=(lx, my, lz),
                    device_id_type=pl.DeviceIdType.MESH,
                )
                bwd.start()
            fwd.wait()
            orig_f = lax.rem(p - h - 1 + NXZ, NXZ)
            out_ref[pl.ds(orig_f * rb, rb), :] = fwd_buf[h + 1].astype(
                jnp.float32
            )
            if h < BHOP:
                bwd.wait()
                orig_b = lax.rem(p + h + 1, NXZ)
                out_ref[pl.ds(orig_b * rb, rb), :] = bwd_buf[h + 1].astype(
                    jnp.float32
                )

    return pl.pallas_call(
        body,
        out_shape=jax.ShapeDtypeStruct((m, n), jnp.float32),
        in_specs=[pl.BlockSpec(memory_space=pltpu.VMEM)],
        out_specs=pl.BlockSpec(memory_space=pltpu.VMEM),
        scratch_shapes=[
            pltpu.VMEM((YHOP, rb, n), jnp.bfloat16),
            pltpu.VMEM((YHOP, rb, n), jnp.bfloat16),
            pltpu.VMEM((FHOP + 1, rb, n), jnp.bfloat16),
            pltpu.VMEM((BHOP + 1, rb, n), jnp.bfloat16),
            pltpu.SemaphoreType.DMA((FHOP,)),
            pltpu.SemaphoreType.DMA((FHOP,)),
            pltpu.SemaphoreType.DMA((BHOP,)),
            pltpu.SemaphoreType.DMA((BHOP,)),
        ],
        compiler_params=pltpu.CompilerParams(collective_id=0),
    )(x)


# device time: 42886 ns/iter; 1.9888x vs baseline; 1.0722x over previous
import functools

import jax
import jax.numpy as jnp
from jax import lax
from jax.experimental import pallas as pl
from jax.experimental.pallas import tpu as pltpu

PY = 4
NXZ = 8
YPEER = PY - 1
FHOP = 4
BHOP = 3


def _ring_coords(p):
    p = lax.rem(p + NXZ, NXZ)
    x = jnp.where(p < 4, 0, 1)
    z = jnp.where(p < 4, p, 7 - p)
    return x, z


def kernel(x):
    _, m, n_glob = x.shape
    n = n_glob // PY
    rb = m // NXZ

    def body(x_ref, out_ref, ys_buf, yr_buf, fwd_buf, bwd_buf,
             fs_sem, fr_sem, bs_sem, br_sem):
        mx = lax.axis_index("x")
        my = lax.axis_index("y")
        mz = lax.axis_index("z")
        p = jnp.where(mx == 0, mz, 7 - mz)
        rx, rz = _ring_coords(p + 1)
        lx, lz = _ring_coords(p - 1)

        barrier_sem = pltpu.get_barrier_semaphore()
        for d in range(1, PY):
            pl.semaphore_signal(
                barrier_sem, inc=1,
                device_id=(mx, lax.rem(my + d, PY), mz),
                device_id_type=pl.DeviceIdType.MESH,
            )
        for dev in ((lx, my, lz), (rx, my, rz)):
            pl.semaphore_signal(
                barrier_sem, inc=1,
                device_id=dev, device_id_type=pl.DeviceIdType.MESH,
            )
        pl.semaphore_wait(barrier_sem, PY - 1 + 2)

        def chunk(c):
            return x_ref[0, pl.ds(p * rb, rb), pl.ds(c * n, n)]

        sends = []
        for h in range(YPEER):
            c = lax.rem(my + 1 + h, PY)
            s = lax.rem(my - c - 1 + 2 * PY, PY)
            ys_buf[h] = chunk(c).astype(jnp.bfloat16)
            sd = pltpu.make_async_remote_copy(
                src_ref=ys_buf.at[h],
                dst_ref=yr_buf.at[s],
                send_sem=bs_sem.at[h],
                recv_sem=br_sem.at[s],
                device_id=(mx, c, mz),
                device_id_type=pl.DeviceIdType.MESH,
            )
            sd.start()
            sends.append(sd)

        for s in range(YPEER):
            rc = pltpu.make_async_remote_copy(
                src_ref=ys_buf.at[0],
                dst_ref=yr_buf.at[s],
                send_sem=bs_sem.at[0],
                recv_sem=br_sem.at[s],
                device_id=(mx, my, mz),
                device_id_type=pl.DeviceIdType.MESH,
            )
            rc.wait_recv()
        for sd in sends:
            sd.wait_send()

        own = (
            (yr_buf[0].astype(jnp.float32) + yr_buf[1].astype(jnp.float32))
            + (yr_buf[2].astype(jnp.float32) + chunk(my))
        )
        own_bf = own.astype(jnp.bfloat16)
        fwd_buf[0] = own_bf
        bwd_buf[0] = own_bf

        def mk_fwd(h):
            return pltpu.make_async_remote_copy(
                src_ref=fwd_buf.at[h], dst_ref=fwd_buf.at[h + 1],
                send_sem=fs_sem.at[h], recv_sem=fr_sem.at[h],
                device_id=(rx, my, rz),
                device_id_type=pl.DeviceIdType.MESH,
            )

        def mk_bwd(h):
            return pltpu.make_async_remote_copy(
                src_ref=bwd_buf.at[h], dst_ref=bwd_buf.at[h + 1],
                send_sem=bs_sem.at[h], recv_sem=br_sem.at[h],
                device_id=(lx, my, lz),
                device_id_type=pl.DeviceIdType.MESH,
            )

        fwds = [mk_fwd(h) for h in range(FHOP)]
        bwds = [mk_bwd(h) for h in range(BHOP)]

        fwds[0].start()

        @functools.partial(pl.run_scoped, mid_sem=pltpu.SemaphoreType.REGULAR)
        def _(mid_sem):
            pl.semaphore_signal(
                mid_sem, inc=1,
                device_id=(rx, my, rz), device_id_type=pl.DeviceIdType.MESH,
            )
            pl.semaphore_wait(mid_sem, 1)

        bwds[0].start()
        out_ref[pl.ds(p * rb, rb), :] = own

        for h in range(FHOP):
            fwds[h].wait_recv()
            if h + 1 < FHOP:
                fwds[h + 1].start()
            if h < BHOP:
                bwds[h].wait_recv()
                if h + 1 < BHOP:
                    bwds[h + 1].start()
            orig_f = lax.rem(p - h - 1 + NXZ, NXZ)
            out_ref[pl.ds(orig_f * rb, rb), :] = fwd_buf[h + 1].astype(
                jnp.float32
            )
            if h < BHOP:
                orig_b = lax.rem(p + h + 1, NXZ)
                out_ref[pl.ds(orig_b * rb, rb), :] = bwd_buf[h + 1].astype(
                    jnp.float32
                )

        for d in fwds:
            d.wait_send()
        for d in bwds:
            d.wait_send()

    return pl.pallas_call(
        body,
        out_shape=jax.ShapeDtypeStruct((m, n), jnp.float32),
        in_specs=[pl.BlockSpec(memory_space=pltpu.VMEM)],
        out_specs=pl.BlockSpec(memory_space=pltpu.VMEM),
        scratch_shapes=[
            pltpu.VMEM((YPEER, rb, n), jnp.bfloat16),
            pltpu.VMEM((YPEER, rb, n), jnp.bfloat16),
            pltpu.VMEM((FHOP + 1, rb, n), jnp.bfloat16),
            pltpu.VMEM((BHOP + 1, rb, n), jnp.bfloat16),
            pltpu.SemaphoreType.DMA((FHOP,)),
            pltpu.SemaphoreType.DMA((FHOP,)),
            pltpu.SemaphoreType.DMA((BHOP,)),
            pltpu.SemaphoreType.DMA((BHOP,)),
        ],
        compiler_params=pltpu.CompilerParams(collective_id=0),
    )(x)


# device time: 42412 ns/iter; 2.0110x vs baseline; 1.0112x over previous
import functools

import jax
import jax.numpy as jnp
from jax import lax
from jax.experimental import pallas as pl
from jax.experimental.pallas import tpu as pltpu

PY = 4
NXZ = 8
YPEER = PY - 1
FHOP = 3
BHOP = 3


def _ring_coords(p):
    p = lax.rem(p + NXZ, NXZ)
    x = jnp.where(p < 4, 0, 1)
    z = jnp.where(p < 4, p, 7 - p)
    return x, z


def kernel(x):
    _, m, n_glob = x.shape
    n = n_glob // PY
    rb = m // NXZ

    def body(x_ref, out_ref, ys_buf, yr_buf, fwd_buf, bwd_buf, ap_buf,
             fs_sem, fr_sem, bs_sem, br_sem, as_sem, ar_sem):
        mx = lax.axis_index("x")
        my = lax.axis_index("y")
        mz = lax.axis_index("z")
        p = jnp.where(mx == 0, mz, 7 - mz)
        rx, rz = _ring_coords(p + 1)
        lx, lz = _ring_coords(p - 1)
        ax, az = _ring_coords(p + 4)

        barrier_sem = pltpu.get_barrier_semaphore()
        for d in range(1, PY):
            pl.semaphore_signal(
                barrier_sem, inc=1,
                device_id=(mx, lax.rem(my + d, PY), mz),
                device_id_type=pl.DeviceIdType.MESH,
            )
        for dev in ((lx, my, lz), (rx, my, rz), (ax, my, az)):
            pl.semaphore_signal(
                barrier_sem, inc=1,
                device_id=dev, device_id_type=pl.DeviceIdType.MESH,
            )

        def chunk(c):
            return x_ref[0, pl.ds(p * rb, rb), pl.ds(c * n, n)]

        for h in range(YPEER):
            c = lax.rem(my + 1 + h, PY)
            ys_buf[h] = chunk(c).astype(jnp.bfloat16)

        pl.semaphore_wait(barrier_sem, PY - 1 + 3)

        sends = []
        for h in range(YPEER):
            c = lax.rem(my + 1 + h, PY)
            s = lax.rem(my - c - 1 + 2 * PY, PY)
            sd = pltpu.make_async_remote_copy(
                src_ref=ys_buf.at[h],
                dst_ref=yr_buf.at[s],
                send_sem=bs_sem.at[h],
                recv_sem=br_sem.at[s],
                device_id=(mx, c, mz),
                device_id_type=pl.DeviceIdType.MESH,
            )
            sd.start()
            sends.append(sd)

        acc = chunk(my)
        for s in range(YPEER):
            rc = pltpu.make_async_remote_copy(
                src_ref=ys_buf.at[0],
                dst_ref=yr_buf.at[s],
                send_sem=bs_sem.at[0],
                recv_sem=br_sem.at[s],
                device_id=(mx, my, mz),
                device_id_type=pl.DeviceIdType.MESH,
            )
            rc.wait_recv()
            acc = acc + yr_buf[s].astype(jnp.float32)
        for sd in sends:
            sd.wait_send()

        own = acc
        own_bf = own.astype(jnp.bfloat16)
        fwd_buf[0] = own_bf
        bwd_buf[0] = own_bf

        def mk_fwd(h):
            return pltpu.make_async_remote_copy(
                src_ref=fwd_buf.at[h], dst_ref=fwd_buf.at[h + 1],
                send_sem=fs_sem.at[h], recv_sem=fr_sem.at[h],
                device_id=(rx, my, rz),
                device_id_type=pl.DeviceIdType.MESH,
            )

        def mk_bwd(h):
            return pltpu.make_async_remote_copy(
                src_ref=bwd_buf.at[h], dst_ref=bwd_buf.at[h + 1],
                send_sem=bs_sem.at[h], recv_sem=br_sem.at[h],
                device_id=(lx, my, lz),
                device_id_type=pl.DeviceIdType.MESH,
            )

        fwds = [mk_fwd(h) for h in range(FHOP)]
        bwds = [mk_bwd(h) for h in range(BHOP)]
        ap = pltpu.make_async_remote_copy(
            src_ref=fwd_buf.at[0], dst_ref=ap_buf,
            send_sem=as_sem, recv_sem=ar_sem,
            device_id=(ax, my, az),
            device_id_type=pl.DeviceIdType.MESH,
        )

        fwds[0].start()
        ap.start()

        @functools.partial(pl.run_scoped, mid_sem=pltpu.SemaphoreType.REGULAR)
        def _(mid_sem):
            pl.semaphore_signal(
                mid_sem, inc=1,
                device_id=(rx, my, rz), device_id_type=pl.DeviceIdType.MESH,
            )
            pl.semaphore_wait(mid_sem, 1)

        bwds[0].start()
        out_ref[pl.ds(p * rb, rb), :] = own

        for h in range(FHOP):
            fwds[h].wait_recv()
            if h + 1 < FHOP:
                fwds[h + 1].start()
            if h < BHOP:
                bwds[h].wait_recv()
                if h + 1 < BHOP:
                    bwds[h + 1].start()
            orig_f = lax.rem(p - h - 1 + NXZ, NXZ)
            out_ref[pl.ds(orig_f * rb, rb), :] = fwd_buf[h + 1].astype(
                jnp.float32
            )
            if h < BHOP:
                orig_b = lax.rem(p + h + 1, NXZ)
                out_ref[pl.ds(orig_b * rb, rb), :] = bwd_buf[h + 1].astype(
                    jnp.float32
                )

        ap.wait_recv()
        orig_a = lax.rem(p + 4, NXZ)
        out_ref[pl.ds(orig_a * rb, rb), :] = ap_buf[...].astype(jnp.float32)

        for d in fwds:
            d.wait_send()
        for d in bwds:
            d.wait_send()
        ap.wait_send()

    return pl.pallas_call(
        body,
        out_shape=jax.ShapeDtypeStruct((m, n), jnp.float32),
        in_specs=[pl.BlockSpec(memory_space=pltpu.VMEM)],
        out_specs=pl.BlockSpec(memory_space=pltpu.VMEM),
        scratch_shapes=[
            pltpu.VMEM((YPEER, rb, n), jnp.bfloat16),
            pltpu.VMEM((YPEER, rb, n), jnp.bfloat16),
            pltpu.VMEM((FHOP + 1, rb, n), jnp.bfloat16),
            pltpu.VMEM((BHOP + 1, rb, n), jnp.bfloat16),
            pltpu.VMEM((rb, n), jnp.bfloat16),
            pltpu.SemaphoreType.DMA((FHOP,)),
            pltpu.SemaphoreType.DMA((FHOP,)),
            pltpu.SemaphoreType.DMA((BHOP,)),
            pltpu.SemaphoreType.DMA((BHOP,)),
            pltpu.SemaphoreType.DMA,
            pltpu.SemaphoreType.DMA,
        ],
        compiler_params=pltpu.CompilerParams(collective_id=0),
    )(x)
